# baseline (speedup 1.0000x reference)
import functools

import numpy as np
import jax
import jax.numpy as jnp
from jax import lax
from jax.experimental import pallas as pl
from jax.experimental.pallas import tpu as pltpu
from jax.experimental.pallas import tpu_sc as plsc

_NATMS = 64
_R = 60
_BLK = _R * _R
_NE = _NATMS * (_NATMS - 1)
_LANES = 16
_NVEC = _BLK // _LANES
_NW = 32
_C = 8
_NCHPA = _NATMS // _C
_PARTS = 4
_APP = _NATMS // _PARTS
_NCH = _APP * _NCHPA // _NW


def _perm_index() -> np.ndarray:
    irreps = [(16, 0), (8, 1), (4, 2)]
    m_idx_map = {0: [0], 1: [2, 0, 1], 2: [0, 1, 2, 3, 4]}
    offsets = [0]
    for mul, l in irreps:
        offsets.append(offsets[-1] + mul * (2 * l + 1))
    rmap = np.zeros(_R, dtype=np.int64)
    src = 0
    for (mul, l), base in zip(irreps, offsets):
        off = 0
        for _ in range(mul):
            for mq in range(2 * l + 1):
                rmap[src] = base + off + m_idx_map[l][mq]
                src += 1
            off += 2 * l + 1
    return (rmap[:, None] * _R + rmap[None, :]).reshape(-1).astype(np.int32)


_PIDX = _perm_index()


def _make_assemble(base: int):
    @functools.partial(
        pl.kernel,
        out_type=jax.ShapeDtypeStruct((_APP * _NATMS * _BLK,), jnp.float32),
        mesh=plsc.VectorSubcoreMesh(core_axis_name="c", subcore_axis_name="s"),
        compiler_params=pltpu.CompilerParams(
            needs_layout_passes=False, use_tc_tiling_on_sc=False),
        scratch_types=[
            pltpu.VMEM((_BLK,), jnp.int32),
            pltpu.VMEM((_C * _BLK,), jnp.float32),
            pltpu.VMEM((_C * _BLK,), jnp.float32),
            pltpu.VMEM((_C * _BLK,), jnp.float32),
            pltpu.VMEM((_C * _BLK,), jnp.float32),
            pltpu.SemaphoreType.DMA,
            pltpu.SemaphoreType.DMA,
            pltpu.SemaphoreType.DMA,
            pltpu.SemaphoreType.DMA,
        ],
    )
    def _assemble(node_hbm, edge_hbm, pidx_hbm, out_hbm, pidx_v,
                  ibuf0, ibuf1, obuf0, obuf1, sin0, sin1, sout0, sout1):
        _assemble_body(base, node_hbm, edge_hbm, pidx_hbm, out_hbm, pidx_v,
                       (ibuf0, ibuf1), (obuf0, obuf1),
                       (sin0, sin1), (sout0, sout1))
    return _assemble


def _assemble_body(base, node_hbm, edge_hbm, pidx_hbm, out_hbm, pidx_v,
                   ibufs, obufs, sins, souts):
    wid = lax.axis_index("s") * 2 + lax.axis_index("c")
    pltpu.sync_copy(pidx_hbm, pidx_v)
    g0 = wid * _NCH

    def chunk_params(t):
        g = g0 + t
        a = base + g // _NCHPA
        b0 = (g % _NCHPA) * _C
        e0 = a * (_NATMS - 1) + b0 - jnp.where(b0 > a, 1, 0)
        is_diag = jnp.logical_and(b0 <= a, a < b0 + _C)
        return a, b0, e0, is_diag

    def issue_in(t, par):
        a, b0, e0, is_diag = chunk_params(t)
        ib = ibufs[par]
        sem = sins[par]

        @pl.when(is_diag)
        def _():
            pltpu.async_copy(edge_hbm.at[pl.ds(e0 * _BLK, (_C - 1) * _BLK)],
                             ib.at[pl.ds(0, (_C - 1) * _BLK)], sem)
            pltpu.async_copy(node_hbm.at[pl.ds(a * _BLK, _BLK)],
                             ib.at[pl.ds((_C - 1) * _BLK, _BLK)], sem)

        @pl.when(jnp.logical_not(is_diag))
        def _():
            pltpu.async_copy(edge_hbm.at[pl.ds(e0 * _BLK, _C * _BLK)], ib, sem)

    issue_in(0, 0)
    issue_in(1, 1)

    def outer(o, carry):
        tt = o * 2
        for par in range(2):
            t = tt + par
            a, b0, e0, is_diag = chunk_params(t)
            ib = ibufs[par]
            ob = obufs[par]
            pltpu.make_async_copy(edge_hbm.at[pl.ds(0, _C * _BLK)], ib,
                                  sins[par]).wait()

            @pl.when(t >= 2)
            def _():
                pltpu.make_async_copy(ob, out_hbm.at[pl.ds(0, _C * _BLK)],
                                      souts[par]).wait()

            p_d = jnp.where(is_diag, a - b0, 2 * _C)
            bases = [
                (jnp.where(p == p_d, _C - 1, p - (p > p_d)) * _BLK).astype(jnp.int32)
                for p in range(_C)
            ]

            @plsc.parallel_loop(0, _NVEC, 1, unroll=4)
            def _(c):
                off = c * _LANES
                idx = pidx_v[pl.ds(off, _LANES)]
                for p in range(_C):
                    ob[pl.ds(p * _BLK + off, _LANES)] = plsc.load_gather(
                        ib.at[pl.ds(bases[p], _BLK)], [idx])

            pltpu.async_copy(ob, out_hbm.at[pl.ds(((a - base) * _NATMS + b0) * _BLK,
                                                  _C * _BLK)], souts[par])

            @pl.when(t + 2 < _NCH)
            def _():
                issue_in(t + 2, par)
        return carry

    lax.fori_loop(0, _NCH // 2, outer, 0)
    for par in range(2):
        pltpu.make_async_copy(obufs[par], out_hbm.at[pl.ds(0, _C * _BLK)],
                              souts[par]).wait()


_ASSEMBLE_PARTS = [_make_assemble(p * _APP) for p in range(_PARTS)]


def kernel(res_node, res_edge, raw_node_mask, raw_edge_mask, atomic_numbers, edge_index):
    node1 = res_node.reshape(_NATMS * _BLK)
    edge1 = res_edge.reshape(_NE * _BLK)
    pidx = jnp.asarray(_PIDX)
    parts = [f(node1, edge1, pidx) for f in _ASSEMBLE_PARTS]
    rows = _APP * _R
    out = jnp.zeros((_NATMS * _R, _NATMS * _R), jnp.float32)
    for k, p in enumerate(parts):
        out = lax.dynamic_update_slice(out, p.reshape(rows, _NATMS * _R),
                                       (k * rows, 0))
    return out


# speedup vs baseline: 1.1755x; 1.1755x over previous
import functools

import numpy as np
import jax
import jax.numpy as jnp
from jax import lax
from jax.experimental import pallas as pl
from jax.experimental.pallas import tpu as pltpu
from jax.experimental.pallas import tpu_sc as plsc

_NATMS = 64
_R = 60
_BLK = _R * _R
_NE = _NATMS * (_NATMS - 1)
_LANES = 16
_NVEC = _BLK // _LANES
_NW = 32
_C = 8
_NCHPA = _NATMS // _C
_PARTS = 1
_APP = _NATMS // _PARTS
_NCH = _APP * _NCHPA // _NW
_RT_ROWS = 64


def _perm_index() -> np.ndarray:
    irreps = [(16, 0), (8, 1), (4, 2)]
    m_idx_map = {0: [0], 1: [2, 0, 1], 2: [0, 1, 2, 3, 4]}
    offsets = [0]
    for mul, l in irreps:
        offsets.append(offsets[-1] + mul * (2 * l + 1))
    rmap = np.zeros(_R, dtype=np.int64)
    src = 0
    for (mul, l), base in zip(irreps, offsets):
        off = 0
        for _ in range(mul):
            for mq in range(2 * l + 1):
                rmap[src] = base + off + m_idx_map[l][mq]
                src += 1
            off += 2 * l + 1
    return (rmap[:, None] * _R + rmap[None, :]).reshape(-1).astype(np.int32)


_PIDX = _perm_index()


def _make_assemble(base: int):
    @functools.partial(
        pl.kernel,
        out_type=jax.ShapeDtypeStruct((_APP * _NATMS * _BLK,), jnp.float32),
        mesh=plsc.VectorSubcoreMesh(core_axis_name="c", subcore_axis_name="s"),
        compiler_params=pltpu.CompilerParams(
            needs_layout_passes=False, use_tc_tiling_on_sc=False),
        scratch_types=[
            pltpu.VMEM((_BLK,), jnp.int32),
            pltpu.VMEM((_C * _BLK,), jnp.float32),
            pltpu.VMEM((_C * _BLK,), jnp.float32),
            pltpu.VMEM((_C * _BLK,), jnp.float32),
            pltpu.VMEM((_C * _BLK,), jnp.float32),
            pltpu.SemaphoreType.DMA,
            pltpu.SemaphoreType.DMA,
            pltpu.SemaphoreType.DMA,
            pltpu.SemaphoreType.DMA,
        ],
    )
    def _assemble(node_hbm, edge_hbm, pidx_hbm, out_hbm, pidx_v,
                  ibuf0, ibuf1, obuf0, obuf1, sin0, sin1, sout0, sout1):
        _assemble_body(base, node_hbm, edge_hbm, pidx_hbm, out_hbm, pidx_v,
                       (ibuf0, ibuf1), (obuf0, obuf1),
                       (sin0, sin1), (sout0, sout1))
    return _assemble


def _assemble_body(base, node_hbm, edge_hbm, pidx_hbm, out_hbm, pidx_v,
                   ibufs, obufs, sins, souts):
    wid = lax.axis_index("s") * 2 + lax.axis_index("c")
    pltpu.sync_copy(pidx_hbm, pidx_v)
    g0 = wid * _NCH

    def chunk_params(t):
        g = g0 + t
        a = base + g // _NCHPA
        b0 = (g % _NCHPA) * _C
        e0 = a * (_NATMS - 1) + b0 - jnp.where(b0 > a, 1, 0)
        is_diag = jnp.logical_and(b0 <= a, a < b0 + _C)
        return a, b0, e0, is_diag

    def issue_in(t, par):
        a, b0, e0, is_diag = chunk_params(t)
        ib = ibufs[par]
        sem = sins[par]

        @pl.when(is_diag)
        def _():
            pltpu.async_copy(edge_hbm.at[pl.ds(e0 * _BLK, (_C - 1) * _BLK)],
                             ib.at[pl.ds(0, (_C - 1) * _BLK)], sem)
            pltpu.async_copy(node_hbm.at[pl.ds(a * _BLK, _BLK)],
                             ib.at[pl.ds((_C - 1) * _BLK, _BLK)], sem)

        @pl.when(jnp.logical_not(is_diag))
        def _():
            pltpu.async_copy(edge_hbm.at[pl.ds(e0 * _BLK, _C * _BLK)], ib, sem)

    issue_in(0, 0)
    issue_in(1, 1)

    def outer(o, carry):
        tt = o * 2
        for par in range(2):
            t = tt + par
            a, b0, e0, is_diag = chunk_params(t)
            ib = ibufs[par]
            ob = obufs[par]
            pltpu.make_async_copy(edge_hbm.at[pl.ds(0, _C * _BLK)], ib,
                                  sins[par]).wait()

            @pl.when(t >= 2)
            def _():
                pltpu.make_async_copy(ob, out_hbm.at[pl.ds(0, _C * _BLK)],
                                      souts[par]).wait()

            p_d = jnp.where(is_diag, a - b0, 2 * _C)
            bases = [
                (jnp.where(p == p_d, _C - 1, p - (p > p_d)) * _BLK).astype(jnp.int32)
                for p in range(_C)
            ]

            @plsc.parallel_loop(0, _NVEC, 1, unroll=4)
            def _(c):
                off = c * _LANES
                idx = pidx_v[pl.ds(off, _LANES)]
                for p in range(_C):
                    ob[pl.ds(p * _BLK + off, _LANES)] = plsc.load_gather(
                        ib.at[pl.ds(bases[p], _BLK)], [idx])

            pltpu.async_copy(ob, out_hbm.at[pl.ds(((a - base) * _NATMS + b0) * _BLK,
                                                  _C * _BLK)], souts[par])

            @pl.when(t + 2 < _NCH)
            def _():
                issue_in(t + 2, par)
        return carry

    lax.fori_loop(0, _NCH // 2, outer, 0)
    for par in range(2):
        pltpu.make_async_copy(obufs[par], out_hbm.at[pl.ds(0, _C * _BLK)],
                              souts[par]).wait()


_ASSEMBLE_PARTS = [_make_assemble(p * _APP) for p in range(_PARTS)]

_N = _NATMS * _R


def _retile_body(in_ref, out_ref):
    for t in range(_N // 128):
        out_ref[:, t * 128:(t + 1) * 128] = in_ref[
            pl.Slice(t, _RT_ROWS, _N // 128), :]


_retile = pl.pallas_call(
    _retile_body,
    grid=(_N // _RT_ROWS,),
    in_specs=[pl.BlockSpec((_RT_ROWS * _N // 128, 128), lambda i: (i, 0))],
    out_specs=pl.BlockSpec((_RT_ROWS, _N), lambda i: (i, 0)),
    out_shape=jax.ShapeDtypeStruct((_N, _N), jnp.float32),
)


def kernel(res_node, res_edge, raw_node_mask, raw_edge_mask, atomic_numbers, edge_index):
    node1 = res_node.reshape(_NATMS * _BLK)
    edge1 = res_edge.reshape(_NE * _BLK)
    pidx = jnp.asarray(_PIDX)
    flat = _ASSEMBLE_PARTS[0](node1, edge1, pidx)
    return _retile(flat.reshape(_N * _N // 128, 128))


# speedup vs baseline: 1.1863x; 1.0093x over previous
import functools

import numpy as np
import jax
import jax.numpy as jnp
from jax import lax
from jax.experimental import pallas as pl
from jax.experimental.pallas import tpu as pltpu
from jax.experimental.pallas import tpu_sc as plsc

_NATMS = 64
_R = 60
_BLK = _R * _R
_NE = _NATMS * (_NATMS - 1)
_LANES = 16
_NVEC = _BLK // _LANES
_NW = 32
_ATOMS_PER_W = _NATMS // _NW
_C = 8
_NCHPA = _NATMS // _C
_NCH = _ATOMS_PER_W * _NCHPA


def _perm_index() -> np.ndarray:
    irreps = [(16, 0), (8, 1), (4, 2)]
    m_idx_map = {0: [0], 1: [2, 0, 1], 2: [0, 1, 2, 3, 4]}
    offsets = [0]
    for mul, l in irreps:
        offsets.append(offsets[-1] + mul * (2 * l + 1))
    rmap = np.zeros(_R, dtype=np.int64)
    src = 0
    for (mul, l), base in zip(irreps, offsets):
        off = 0
        for _ in range(mul):
            for mq in range(2 * l + 1):
                rmap[src] = base + off + m_idx_map[l][mq]
                src += 1
            off += 2 * l + 1
    return (rmap[:, None] * _R + rmap[None, :]).reshape(-1).astype(np.int32)


_PIDX = _perm_index()


@functools.partial(
    pl.kernel,
    out_type=jax.ShapeDtypeStruct((_NATMS * _NATMS * _BLK,), jnp.float32),
    mesh=plsc.VectorSubcoreMesh(core_axis_name="c", subcore_axis_name="s"),
    compiler_params=pltpu.CompilerParams(
        needs_layout_passes=False, use_tc_tiling_on_sc=False),
    scratch_types=[
        pltpu.VMEM((_BLK,), jnp.int32),
        pltpu.VMEM((_C * _BLK,), jnp.float32),
        pltpu.VMEM((_C * _BLK,), jnp.float32),
        pltpu.VMEM((_C * _BLK,), jnp.float32),
        pltpu.VMEM((_C * _BLK,), jnp.float32),
        pltpu.SemaphoreType.DMA,
        pltpu.SemaphoreType.DMA,
        pltpu.SemaphoreType.DMA,
        pltpu.SemaphoreType.DMA,
    ],
)
def _assemble(node_hbm, edge_hbm, pidx_hbm, out_hbm, pidx_v,
              ibuf0, ibuf1, obuf0, obuf1, sin0, sin1, sout0, sout1):
    wid = lax.axis_index("s") * 2 + lax.axis_index("c")
    pltpu.sync_copy(pidx_hbm, pidx_v)
    ibufs = (ibuf0, ibuf1)
    obufs = (obuf0, obuf1)
    sins = (sin0, sin1)
    souts = (sout0, sout1)
    a0 = wid * _ATOMS_PER_W

    def chunk_params(t):
        a = a0 + t // _NCHPA
        b0 = (t % _NCHPA) * _C
        e0 = a * (_NATMS - 1) + b0 - jnp.where(b0 > a, 1, 0)
        is_diag = jnp.logical_and(b0 <= a, a < b0 + _C)
        return a, b0, e0, is_diag

    def issue_in(t, par):
        a, b0, e0, is_diag = chunk_params(t)
        ib = ibufs[par]
        sem = sins[par]

        @pl.when(is_diag)
        def _():
            pltpu.async_copy(edge_hbm.at[pl.ds(e0 * _BLK, (_C - 1) * _BLK)],
                             ib.at[pl.ds(0, (_C - 1) * _BLK)], sem)
            pltpu.async_copy(node_hbm.at[pl.ds(a * _BLK, _BLK)],
                             ib.at[pl.ds((_C - 1) * _BLK, _BLK)], sem)

        @pl.when(jnp.logical_not(is_diag))
        def _():
            pltpu.async_copy(edge_hbm.at[pl.ds(e0 * _BLK, _C * _BLK)], ib, sem)

    issue_in(0, 0)
    issue_in(1, 1)

    def outer(o, carry):
        tt = o * 2
        for par in range(2):
            t = tt + par
            a, b0, e0, is_diag = chunk_params(t)
            ib = ibufs[par]
            ob = obufs[par]
            pltpu.make_async_copy(edge_hbm.at[pl.ds(0, _C * _BLK)], ib,
                                  sins[par]).wait()

            @pl.when(t >= 2)
            def _():
                pltpu.make_async_copy(ob, out_hbm.at[pl.ds(0, _C * _BLK)],
                                      souts[par]).wait()

            p_d = jnp.where(is_diag, a - b0, 2 * _C)
            bases = [
                (jnp.where(p == p_d, _C - 1, p - (p > p_d)) * _BLK).astype(jnp.int32)
                for p in range(_C)
            ]

            @plsc.parallel_loop(0, _NVEC, 1, unroll=4)
            def _(c):
                off = c * _LANES
                idx = pidx_v[pl.ds(off, _LANES)]
                for p in range(_C):
                    ob[pl.ds(p * _BLK + off, _LANES)] = plsc.load_gather(
                        ib.at[pl.ds(bases[p], _BLK)], [idx])

            pltpu.async_copy(ob, out_hbm.at[pl.ds((a * _NATMS + b0) * _BLK,
                                                  _C * _BLK)], souts[par])

            @pl.when(t + 2 < _NCH)
            def _():
                issue_in(t + 2, par)
        return carry

    lax.fori_loop(0, _NCH // 2, outer, 0)
    for par in range(2):
        pltpu.make_async_copy(obufs[par], out_hbm.at[pl.ds(0, _C * _BLK)],
                              souts[par]).wait()


def kernel(res_node, res_edge, raw_node_mask, raw_edge_mask, atomic_numbers, edge_index):
    node1 = res_node.reshape(_NATMS * _BLK)
    edge1 = res_edge.reshape(_NE * _BLK)
    out = _assemble(node1, edge1, jnp.asarray(_PIDX))
    return out.reshape(_NATMS * _R, _NATMS * _R)
